# baseline (speedup 1.0000x reference)
import functools

import jax
import jax.numpy as jnp
from jax import lax
from jax.experimental import pallas as pl
from jax.experimental.pallas import tpu as pltpu
from jax.experimental.pallas import tpu_sc as plsc

_N = 10000
_E = 320000
_D = 128
_NC = 2
_NS = 16
_K = 80
_C = 128
_EP = _NC * _NS * _K * _C
_KH = _K // 2
_NP = 10112
_RPT = _NP // _NS

_BLK = 2000


def _mlp_in_body(x_ref, w_ref, b_ref, o_ref):
    o_ref[...] = jnp.tanh(
        jnp.dot(x_ref[...], w_ref[...], preferred_element_type=jnp.float32)
        + b_ref[...])


def _mlp_in(x, W, b):
    return pl.pallas_call(
        _mlp_in_body,
        grid=(_N // _BLK,),
        in_specs=[
            pl.BlockSpec((_BLK, _D), lambda i: (i, 0)),
            pl.BlockSpec((_D, _D), lambda i: (0, 0)),
            pl.BlockSpec((1, _D), lambda i: (0, 0)),
        ],
        out_specs=pl.BlockSpec((_BLK, _D), lambda i: (i, 0)),
        out_shape=jax.ShapeDtypeStruct((_N, _D), jnp.float32),
    )(x, W, b)


def _edge_body(h_ref, w1_ref, b1_ref, w2_ref, b2_ref, o_ref):
    t = jnp.tanh(
        jnp.dot(h_ref[...], w1_ref[...], preferred_element_type=jnp.float32)
        + b1_ref[...])
    o_ref[...] = jnp.dot(t, w2_ref[...],
                         preferred_element_type=jnp.float32) + b2_ref[...]


def _edge_mlp(h, W1, b1, W2, b2):
    return pl.pallas_call(
        _edge_body,
        grid=(_N // _BLK,),
        in_specs=[
            pl.BlockSpec((_BLK, _D), lambda i: (i, 0)),
            pl.BlockSpec((_D, _D), lambda i: (0, 0)),
            pl.BlockSpec((1, _D), lambda i: (0, 0)),
            pl.BlockSpec((_D, _D), lambda i: (0, 0)),
            pl.BlockSpec((1, _D), lambda i: (0, 0)),
        ],
        out_specs=pl.BlockSpec((_BLK, _D), lambda i: (i, 0)),
        out_shape=jax.ShapeDtypeStruct((_N, _D), jnp.float32),
    )(h, W1, b1, W2, b2)


def _node_body(h_ref, p_ref, wh_ref, wm_ref, b_ref, o_ref):
    m = p_ref[0] + p_ref[1]
    o_ref[...] = jnp.tanh(
        jnp.dot(h_ref[...], wh_ref[...], preferred_element_type=jnp.float32)
        + jnp.dot(m, wm_ref[...], preferred_element_type=jnp.float32)
        + b_ref[...])


def _node_mlp(h, parts, Wh, Wm, b):
    return pl.pallas_call(
        _node_body,
        grid=(_N // _BLK,),
        in_specs=[
            pl.BlockSpec((_BLK, _D), lambda i: (i, 0)),
            pl.BlockSpec((_NC, _BLK, _D), lambda i: (0, i, 0)),
            pl.BlockSpec((_D, _D), lambda i: (0, 0)),
            pl.BlockSpec((_D, _D), lambda i: (0, 0)),
            pl.BlockSpec((1, _D), lambda i: (0, 0)),
        ],
        out_specs=pl.BlockSpec((_BLK, _D), lambda i: (i, 0)),
        out_shape=jax.ShapeDtypeStruct((_N, _D), jnp.float32),
    )(h, parts, Wh, Wm, b)


def _out_body(h_ref, w_ref, b_ref, o_ref):
    o_ref[...] = jnp.dot(h_ref[...], w_ref[...],
                         preferred_element_type=jnp.float32) + b_ref[...]


def _out_proj(h, W, b):
    return pl.pallas_call(
        _out_body,
        grid=(_N // _BLK,),
        in_specs=[
            pl.BlockSpec((_BLK, _D), lambda i: (i, 0)),
            pl.BlockSpec((_D, 1), lambda i: (0, 0)),
            pl.BlockSpec((1, 1), lambda i: (0, 0)),
        ],
        out_specs=pl.BlockSpec((_BLK, 1), lambda i: (i, 0)),
        out_shape=jax.ShapeDtypeStruct((_N, 1), jnp.float32),
    )(h, W, b)


def _sc_body(mall_hbm, src_hbm, dst_hbm, zeros_hbm, out_hbm,
             src_v, dst_v, rows_a, rows_b, acc, sem_a, sem_b):
    c = lax.axis_index("c")
    s = lax.axis_index("s")
    pltpu.sync_copy(zeros_hbm.at[pl.ds(s * _RPT, _RPT)],
                    acc.at[pl.ds(s * _RPT, _RPT)])
    plsc.subcore_barrier()

    for p in range(_K // _KH):
        pltpu.sync_copy(src_hbm.at[c, s, pl.ds(p * _KH, _KH)], src_v)
        pltpu.sync_copy(dst_hbm.at[c, s, pl.ds(p * _KH, _KH)], dst_v)
        pltpu.async_copy(mall_hbm.at[src_v.at[0]], rows_a, sem_a)

        def body(gg, carry):
            g = gg * 2
            pltpu.async_copy(mall_hbm.at[src_v.at[g + 1]], rows_b, sem_b)
            pltpu.make_async_copy(mall_hbm.at[src_v.at[g]], rows_a,
                                  sem_a).wait()
            pltpu.sync_copy(rows_a, acc.at[dst_v.at[g]], add=True)

            @pl.when(g + 2 < _KH)
            def _():
                pltpu.async_copy(mall_hbm.at[src_v.at[g + 2]], rows_a, sem_a)

            pltpu.make_async_copy(mall_hbm.at[src_v.at[g + 1]], rows_b,
                                  sem_b).wait()
            pltpu.sync_copy(rows_b, acc.at[dst_v.at[g + 1]], add=True)
            return carry

        lax.fori_loop(0, _KH // 2, body, 0)
    plsc.subcore_barrier()
    pltpu.sync_copy(acc.at[pl.ds(s * _RPT, _RPT)],
                    out_hbm.at[c, pl.ds(s * _RPT, _RPT)])


def _sc_segsum(m_all, src, dst, zeros):
    mesh = plsc.VectorSubcoreMesh(core_axis_name="c", subcore_axis_name="s")
    f = functools.partial(
        pl.kernel,
        out_type=jax.ShapeDtypeStruct((_NC, _NP, _D), jnp.float32),
        mesh=mesh,
        scratch_types=[
            pltpu.VMEM((_KH, _C), jnp.int32),
            pltpu.VMEM((_KH, _C), jnp.int32),
            pltpu.VMEM((_C, _D), jnp.float32),
            pltpu.VMEM((_C, _D), jnp.float32),
            pltpu.VMEM_SHARED((_NP, _D), jnp.float32),
            pltpu.SemaphoreType.DMA,
            pltpu.SemaphoreType.DMA,
        ],
    )(_sc_body)
    return f(m_all, src, dst, zeros)


def kernel(x, edge_index, W_in, b_in, W_e1, b_e1, W_e2, b_e2, W_n, b_n,
           W_out, b_out):
    pad = _EP - _E
    src = jnp.concatenate(
        [edge_index[0], jnp.zeros((pad,), jnp.int32)]).reshape(
            _NC, _NS, _K, _C)
    dst = jnp.concatenate(
        [edge_index[1], jnp.full((pad,), _N, jnp.int32)]).reshape(
            _NC, _NS, _K, _C)
    zeros = jnp.zeros((_NP, _D), jnp.float32)
    b_in2 = b_in.reshape(1, _D)
    b_e12 = b_e1.reshape(1, _D)
    b_e22 = b_e2.reshape(1, _D)
    b_n2 = b_n.reshape(1, _D)
    b_out2 = b_out.reshape(1, 1)
    W_nh = W_n[:_D]
    W_nm = W_n[_D:]

    h = _mlp_in(x, W_in, b_in2)
    for _ in range(2):
        m_all = _edge_mlp(h, W_e1, b_e12, W_e2, b_e22)
        parts = _sc_segsum(m_all, src, dst, zeros)
        h = _node_mlp(h, parts, W_nh, W_nm, b_n2)
    return _out_proj(h, W_out, b_out2)


# speedup vs baseline: 3.2486x; 3.2486x over previous
import functools

import jax
import jax.numpy as jnp
from jax import lax
from jax.experimental import pallas as pl
from jax.experimental.pallas import tpu as pltpu
from jax.experimental.pallas import tpu_sc as plsc

_N = 10000
_E = 320000
_D = 128
_NC = 2
_NS = 16
_K = 80
_C = 128
_EP = _NC * _NS * _K * _C
_KH = _K // 2
_NP = 10112
_RPT = _NP // _NS

_BLK = 2000


def _mlp_in_body(x_ref, w_ref, b_ref, o_ref):
    o_ref[...] = jnp.tanh(
        jnp.dot(x_ref[...], w_ref[...], preferred_element_type=jnp.float32)
        + b_ref[...])


def _mlp_in(x, W, b):
    return pl.pallas_call(
        _mlp_in_body,
        grid=(_N // _BLK,),
        in_specs=[
            pl.BlockSpec((_BLK, _D), lambda i: (i, 0)),
            pl.BlockSpec((_D, _D), lambda i: (0, 0)),
            pl.BlockSpec((1, _D), lambda i: (0, 0)),
        ],
        out_specs=pl.BlockSpec((_BLK, _D), lambda i: (i, 0)),
        out_shape=jax.ShapeDtypeStruct((_N, _D), jnp.float32),
    )(x, W, b)


def _edge_body(h_ref, w1_ref, b1_ref, w2_ref, b2_ref, o_ref):
    t = jnp.tanh(
        jnp.dot(h_ref[...], w1_ref[...], preferred_element_type=jnp.float32)
        + b1_ref[...])
    o_ref[...] = jnp.dot(t, w2_ref[...],
                         preferred_element_type=jnp.float32) + b2_ref[...]


def _edge_mlp(h, W1, b1, W2, b2):
    return pl.pallas_call(
        _edge_body,
        grid=(_N // _BLK,),
        in_specs=[
            pl.BlockSpec((_BLK, _D), lambda i: (i, 0)),
            pl.BlockSpec((_D, _D), lambda i: (0, 0)),
            pl.BlockSpec((1, _D), lambda i: (0, 0)),
            pl.BlockSpec((_D, _D), lambda i: (0, 0)),
            pl.BlockSpec((1, _D), lambda i: (0, 0)),
        ],
        out_specs=pl.BlockSpec((_BLK, _D), lambda i: (i, 0)),
        out_shape=jax.ShapeDtypeStruct((_N, _D), jnp.float32),
    )(h, W1, b1, W2, b2)


def _node_body(h_ref, p_ref, wh_ref, wm_ref, b_ref, o_ref):
    m = p_ref[0] + p_ref[1]
    o_ref[...] = jnp.tanh(
        jnp.dot(h_ref[...], wh_ref[...], preferred_element_type=jnp.float32)
        + jnp.dot(m, wm_ref[...], preferred_element_type=jnp.float32)
        + b_ref[...])


def _node_mlp(h, parts, Wh, Wm, b):
    return pl.pallas_call(
        _node_body,
        grid=(_N // _BLK,),
        in_specs=[
            pl.BlockSpec((_BLK, _D), lambda i: (i, 0)),
            pl.BlockSpec((_NC, _BLK, _D), lambda i: (0, i, 0)),
            pl.BlockSpec((_D, _D), lambda i: (0, 0)),
            pl.BlockSpec((_D, _D), lambda i: (0, 0)),
            pl.BlockSpec((1, _D), lambda i: (0, 0)),
        ],
        out_specs=pl.BlockSpec((_BLK, _D), lambda i: (i, 0)),
        out_shape=jax.ShapeDtypeStruct((_N, _D), jnp.float32),
    )(h, parts, Wh, Wm, b)


def _out_body(h_ref, w_ref, b_ref, o_ref):
    o_ref[...] = jnp.dot(h_ref[...], w_ref[...],
                         preferred_element_type=jnp.float32) + b_ref[...]


def _out_proj(h, W, b):
    return pl.pallas_call(
        _out_body,
        grid=(_N // _BLK,),
        in_specs=[
            pl.BlockSpec((_BLK, _D), lambda i: (i, 0)),
            pl.BlockSpec((_D, 1), lambda i: (0, 0)),
            pl.BlockSpec((1, 1), lambda i: (0, 0)),
        ],
        out_specs=pl.BlockSpec((_BLK, 1), lambda i: (i, 0)),
        out_shape=jax.ShapeDtypeStruct((_N, 1), jnp.float32),
    )(h, W, b)


def _sc_body(mall_hbm, src_hbm, dst_hbm, zeros_hbm, out_hbm,
             src_v, dst_v, rows_a, rows_b, acc, sem_a, sem_b):
    c = lax.axis_index("c")
    s = lax.axis_index("s")
    pltpu.sync_copy(zeros_hbm.at[pl.ds(s * _RPT, _RPT)],
                    acc.at[pl.ds(s * _RPT, _RPT)])
    plsc.subcore_barrier()

    for p in range(_K // _KH):
        pltpu.sync_copy(src_hbm.at[c, s, pl.ds(p * _KH, _KH)], src_v)
        pltpu.sync_copy(dst_hbm.at[c, s, pl.ds(p * _KH, _KH)], dst_v)
        pltpu.async_copy(mall_hbm.at[src_v.at[0]], rows_a, sem_a)

        def body(gg, carry):
            g = gg * 2
            pltpu.async_copy(mall_hbm.at[src_v.at[g + 1]], rows_b, sem_b)
            pltpu.make_async_copy(mall_hbm.at[src_v.at[g]], rows_a,
                                  sem_a).wait()
            pltpu.sync_copy(rows_a, acc.at[dst_v.at[g]], add=True)

            @pl.when(g + 2 < _KH)
            def _():
                pltpu.async_copy(mall_hbm.at[src_v.at[g + 2]], rows_a, sem_a)

            pltpu.make_async_copy(mall_hbm.at[src_v.at[g + 1]], rows_b,
                                  sem_b).wait()
            pltpu.sync_copy(rows_b, acc.at[dst_v.at[g + 1]], add=True)
            return carry

        lax.fori_loop(0, _KH // 2, body, 0)
    plsc.subcore_barrier()
    pltpu.sync_copy(acc.at[pl.ds(s * _RPT, _RPT)],
                    out_hbm.at[c, pl.ds(s * _RPT, _RPT)])


def _sc_segsum(m_all, src, dst, zeros):
    mesh = plsc.VectorSubcoreMesh(core_axis_name="c", subcore_axis_name="s")
    f = functools.partial(
        pl.kernel,
        out_type=jax.ShapeDtypeStruct((_NC, _NP, _D), jnp.float32),
        mesh=mesh,
        scratch_types=[
            pltpu.VMEM((_KH, _C), jnp.int32),
            pltpu.VMEM((_KH, _C), jnp.int32),
            pltpu.VMEM((_C, _D), jnp.float32),
            pltpu.VMEM((_C, _D), jnp.float32),
            pltpu.VMEM_SHARED((_NP, _D), jnp.float32),
            pltpu.SemaphoreType.DMA,
            pltpu.SemaphoreType.DMA,
        ],
    )(_sc_body)
    return f(m_all, src, dst, zeros)


def kernel(x, edge_index, W_in, b_in, W_e1, b_e1, W_e2, b_e2, W_n, b_n,
           W_out, b_out):
    pad = _EP - _E
    pad_ids = lax.iota(jnp.int32, pad)
    src = jnp.concatenate(
        [edge_index[0], pad_ids % _N]).reshape(_NC, _NS, _K, _C)
    dst = jnp.concatenate(
        [edge_index[1], _N + pad_ids % (_NP - _N)]).reshape(
            _NC, _NS, _K, _C)
    zeros = jnp.zeros((_NP, _D), jnp.float32)
    b_in2 = b_in.reshape(1, _D)
    b_e12 = b_e1.reshape(1, _D)
    b_e22 = b_e2.reshape(1, _D)
    b_n2 = b_n.reshape(1, _D)
    b_out2 = b_out.reshape(1, 1)
    W_nh = W_n[:_D]
    W_nm = W_n[_D:]

    h = _mlp_in(x, W_in, b_in2)
    for _ in range(2):
        m_all = _edge_mlp(h, W_e1, b_e12, W_e2, b_e22)
        parts = _sc_segsum(m_all, src, dst, zeros)
        h = _node_mlp(h, parts, W_nh, W_nm, b_n2)
    return _out_proj(h, W_out, b_out2)


# speedup vs baseline: 3.4561x; 1.0638x over previous
import functools

import jax
import jax.numpy as jnp
from jax import lax
from jax.experimental import pallas as pl
from jax.experimental.pallas import tpu as pltpu
from jax.experimental.pallas import tpu_sc as plsc

_N = 10000
_E = 320000
_D = 128
_NC = 2
_NS = 16
_K = 80
_C = 128
_EP = _NC * _NS * _K * _C
_KH = _K // 2
_NP = 10112
_RPT = _NP // _NS

_BLK = 2000


def _edge_of(h, w1_ref, b1_ref, w2_ref, b2_ref):
    t = jnp.tanh(
        jnp.dot(h, w1_ref[...], preferred_element_type=jnp.float32)
        + b1_ref[...])
    return jnp.dot(t, w2_ref[...],
                   preferred_element_type=jnp.float32) + b2_ref[...]


def _in_edge_body(x_ref, wi_ref, bi_ref, w1_ref, b1_ref, w2_ref, b2_ref,
                  h_ref, m_ref):
    h = jnp.tanh(
        jnp.dot(x_ref[...], wi_ref[...], preferred_element_type=jnp.float32)
        + bi_ref[...])
    h_ref[...] = h
    m_ref[...] = _edge_of(h, w1_ref, b1_ref, w2_ref, b2_ref)


def _in_edge(x, Wi, bi, W1, b1, W2, b2):
    w = pl.BlockSpec((_D, _D), lambda i: (0, 0))
    b = pl.BlockSpec((1, _D), lambda i: (0, 0))
    r = pl.BlockSpec((_BLK, _D), lambda i: (i, 0))
    f = jax.ShapeDtypeStruct((_N, _D), jnp.float32)
    return pl.pallas_call(
        _in_edge_body,
        grid=(_N // _BLK,),
        in_specs=[r, w, b, w, b, w, b],
        out_specs=[r, r],
        out_shape=[f, f],
    )(x, Wi, bi, W1, b1, W2, b2)


def _node_of(h_ref, p_ref, wh_ref, wm_ref, b_ref):
    m = p_ref[0] + p_ref[1]
    return jnp.tanh(
        jnp.dot(h_ref[...], wh_ref[...], preferred_element_type=jnp.float32)
        + jnp.dot(m, wm_ref[...], preferred_element_type=jnp.float32)
        + b_ref[...])


def _node_edge_body(h_ref, p_ref, wh_ref, wm_ref, bn_ref,
                    w1_ref, b1_ref, w2_ref, b2_ref, h2_ref, m_ref):
    h2 = _node_of(h_ref, p_ref, wh_ref, wm_ref, bn_ref)
    h2_ref[...] = h2
    m_ref[...] = _edge_of(h2, w1_ref, b1_ref, w2_ref, b2_ref)


def _node_edge(h, parts, Wh, Wm, bn, W1, b1, W2, b2):
    w = pl.BlockSpec((_D, _D), lambda i: (0, 0))
    b = pl.BlockSpec((1, _D), lambda i: (0, 0))
    r = pl.BlockSpec((_BLK, _D), lambda i: (i, 0))
    p = pl.BlockSpec((_NC, _BLK, _D), lambda i: (0, i, 0))
    f = jax.ShapeDtypeStruct((_N, _D), jnp.float32)
    return pl.pallas_call(
        _node_edge_body,
        grid=(_N // _BLK,),
        in_specs=[r, p, w, w, b, w, b, w, b],
        out_specs=[r, r],
        out_shape=[f, f],
    )(h, parts, Wh, Wm, bn, W1, b1, W2, b2)


def _node_out_body(h_ref, p_ref, wh_ref, wm_ref, bn_ref, wo_ref, bo_ref,
                   o_ref):
    h2 = _node_of(h_ref, p_ref, wh_ref, wm_ref, bn_ref)
    o_ref[...] = jnp.dot(h2, wo_ref[...],
                         preferred_element_type=jnp.float32) + bo_ref[...]


def _node_out(h, parts, Wh, Wm, bn, Wo, bo):
    w = pl.BlockSpec((_D, _D), lambda i: (0, 0))
    b = pl.BlockSpec((1, _D), lambda i: (0, 0))
    r = pl.BlockSpec((_BLK, _D), lambda i: (i, 0))
    p = pl.BlockSpec((_NC, _BLK, _D), lambda i: (0, i, 0))
    return pl.pallas_call(
        _node_out_body,
        grid=(_N // _BLK,),
        in_specs=[r, p, w, w, b,
                  pl.BlockSpec((_D, 1), lambda i: (0, 0)),
                  pl.BlockSpec((1, 1), lambda i: (0, 0))],
        out_specs=pl.BlockSpec((_BLK, 1), lambda i: (i, 0)),
        out_shape=jax.ShapeDtypeStruct((_N, 1), jnp.float32),
    )(h, parts, Wh, Wm, bn, Wo, bo)


def _sc_body(mall_hbm, src_hbm, dst_hbm, zeros_hbm, out_hbm,
             src_v, dst_v, rows_a, rows_b, acc, sem_a, sem_b):
    c = lax.axis_index("c")
    s = lax.axis_index("s")
    pltpu.sync_copy(zeros_hbm.at[pl.ds(s * _RPT, _RPT)],
                    acc.at[pl.ds(s * _RPT, _RPT)])
    plsc.subcore_barrier()

    for p in range(_K // _KH):
        pltpu.sync_copy(src_hbm.at[c, s, pl.ds(p * _KH, _KH)], src_v)
        pltpu.sync_copy(dst_hbm.at[c, s, pl.ds(p * _KH, _KH)], dst_v)
        pltpu.async_copy(mall_hbm.at[src_v.at[0]], rows_a, sem_a)

        def body(gg, carry):
            g = gg * 2
            pltpu.async_copy(mall_hbm.at[src_v.at[g + 1]], rows_b, sem_b)
            pltpu.make_async_copy(mall_hbm.at[src_v.at[g]], rows_a,
                                  sem_a).wait()
            pltpu.sync_copy(rows_a, acc.at[dst_v.at[g]], add=True)

            @pl.when(g + 2 < _KH)
            def _():
                pltpu.async_copy(mall_hbm.at[src_v.at[g + 2]], rows_a, sem_a)

            pltpu.make_async_copy(mall_hbm.at[src_v.at[g + 1]], rows_b,
                                  sem_b).wait()
            pltpu.sync_copy(rows_b, acc.at[dst_v.at[g + 1]], add=True)
            return carry

        lax.fori_loop(0, _KH // 2, body, 0)
    plsc.subcore_barrier()
    pltpu.sync_copy(acc.at[pl.ds(s * _RPT, _RPT)],
                    out_hbm.at[c, pl.ds(s * _RPT, _RPT)])


def _sc_segsum(m_all, src, dst, zeros):
    mesh = plsc.VectorSubcoreMesh(core_axis_name="c", subcore_axis_name="s")
    f = functools.partial(
        pl.kernel,
        out_type=jax.ShapeDtypeStruct((_NC, _NP, _D), jnp.float32),
        mesh=mesh,
        scratch_types=[
            pltpu.VMEM((_KH, _C), jnp.int32),
            pltpu.VMEM((_KH, _C), jnp.int32),
            pltpu.VMEM((_C, _D), jnp.float32),
            pltpu.VMEM((_C, _D), jnp.float32),
            pltpu.VMEM_SHARED((_NP, _D), jnp.float32),
            pltpu.SemaphoreType.DMA,
            pltpu.SemaphoreType.DMA,
        ],
    )(_sc_body)
    return f(m_all, src, dst, zeros)


def kernel(x, edge_index, W_in, b_in, W_e1, b_e1, W_e2, b_e2, W_n, b_n,
           W_out, b_out):
    pad = _EP - _E
    pad_ids = lax.iota(jnp.int32, pad)
    src = jnp.concatenate(
        [edge_index[0], pad_ids % _N]).reshape(_NC, _NS, _K, _C)
    dst = jnp.concatenate(
        [edge_index[1], _N + pad_ids % (_NP - _N)]).reshape(
            _NC, _NS, _K, _C)
    zeros = jnp.zeros((_NP, _D), jnp.float32)
    b_in2 = b_in.reshape(1, _D)
    b_e12 = b_e1.reshape(1, _D)
    b_e22 = b_e2.reshape(1, _D)
    b_n2 = b_n.reshape(1, _D)
    b_out2 = b_out.reshape(1, 1)
    W_nh = W_n[:_D]
    W_nm = W_n[_D:]

    h, m_all = _in_edge(x, W_in, b_in2, W_e1, b_e12, W_e2, b_e22)
    parts = _sc_segsum(m_all, src, dst, zeros)
    h, m_all = _node_edge(h, parts, W_nh, W_nm, b_n2,
                          W_e1, b_e12, W_e2, b_e22)
    parts = _sc_segsum(m_all, src, dst, zeros)
    return _node_out(h, parts, W_nh, W_nm, b_n2, W_out, b_out2)


# speedup vs baseline: 3.5322x; 1.0220x over previous
import functools

import jax
import jax.numpy as jnp
from jax import lax
from jax.experimental import pallas as pl
from jax.experimental.pallas import tpu as pltpu
from jax.experimental.pallas import tpu_sc as plsc

_N = 10000
_E = 320000
_D = 128
_NC = 2
_NS = 16
_K = 80
_C = 128
_EP = _NC * _NS * _K * _C
_KH = _K // 2
_NP = 10112
_RPT = _NP // _NS

_BLK = 2000


def _edge_of(h, w1_ref, b1_ref, w2_ref, b2_ref):
    t = jnp.tanh(
        jnp.dot(h, w1_ref[...], preferred_element_type=jnp.float32)
        + b1_ref[...])
    return jnp.dot(t, w2_ref[...],
                   preferred_element_type=jnp.float32) + b2_ref[...]


def _in_edge_body(x_ref, wi_ref, bi_ref, w1_ref, b1_ref, w2_ref, b2_ref,
                  h_ref, m_ref):
    h = jnp.tanh(
        jnp.dot(x_ref[...], wi_ref[...], preferred_element_type=jnp.float32)
        + bi_ref[...])
    h_ref[...] = h
    m_ref[...] = _edge_of(h, w1_ref, b1_ref, w2_ref, b2_ref)


def _in_edge(x, Wi, bi, W1, b1, W2, b2):
    w = pl.BlockSpec((_D, _D), lambda i: (0, 0))
    b = pl.BlockSpec((1, _D), lambda i: (0, 0))
    r = pl.BlockSpec((_BLK, _D), lambda i: (i, 0))
    f = jax.ShapeDtypeStruct((_N, _D), jnp.float32)
    return pl.pallas_call(
        _in_edge_body,
        grid=(_N // _BLK,),
        in_specs=[r, w, b, w, b, w, b],
        out_specs=[r, r],
        out_shape=[f, f],
    )(x, Wi, bi, W1, b1, W2, b2)


def _node_of(h_ref, p_ref, wh_ref, wm_ref, b_ref):
    m = p_ref[0] + p_ref[1]
    return jnp.tanh(
        jnp.dot(h_ref[...], wh_ref[...], preferred_element_type=jnp.float32)
        + jnp.dot(m, wm_ref[...], preferred_element_type=jnp.float32)
        + b_ref[...])


def _node_edge_body(h_ref, p_ref, wh_ref, wm_ref, bn_ref,
                    w1_ref, b1_ref, w2_ref, b2_ref, h2_ref, m_ref):
    h2 = _node_of(h_ref, p_ref, wh_ref, wm_ref, bn_ref)
    h2_ref[...] = h2
    m_ref[...] = _edge_of(h2, w1_ref, b1_ref, w2_ref, b2_ref)


def _node_edge(h, parts, Wh, Wm, bn, W1, b1, W2, b2):
    w = pl.BlockSpec((_D, _D), lambda i: (0, 0))
    b = pl.BlockSpec((1, _D), lambda i: (0, 0))
    r = pl.BlockSpec((_BLK, _D), lambda i: (i, 0))
    p = pl.BlockSpec((_NC, _BLK, _D), lambda i: (0, i, 0))
    f = jax.ShapeDtypeStruct((_N, _D), jnp.float32)
    return pl.pallas_call(
        _node_edge_body,
        grid=(_N // _BLK,),
        in_specs=[r, p, w, w, b, w, b, w, b],
        out_specs=[r, r],
        out_shape=[f, f],
    )(h, parts, Wh, Wm, bn, W1, b1, W2, b2)


def _node_out_body(h_ref, p_ref, wh_ref, wm_ref, bn_ref, wo_ref, bo_ref,
                   o_ref):
    h2 = _node_of(h_ref, p_ref, wh_ref, wm_ref, bn_ref)
    o_ref[...] = jnp.dot(h2, wo_ref[...],
                         preferred_element_type=jnp.float32) + bo_ref[...]


def _node_out(h, parts, Wh, Wm, bn, Wo, bo):
    w = pl.BlockSpec((_D, _D), lambda i: (0, 0))
    b = pl.BlockSpec((1, _D), lambda i: (0, 0))
    r = pl.BlockSpec((_BLK, _D), lambda i: (i, 0))
    p = pl.BlockSpec((_NC, _BLK, _D), lambda i: (0, i, 0))
    return pl.pallas_call(
        _node_out_body,
        grid=(_N // _BLK,),
        in_specs=[r, p, w, w, b,
                  pl.BlockSpec((_D, 1), lambda i: (0, 0)),
                  pl.BlockSpec((1, 1), lambda i: (0, 0))],
        out_specs=pl.BlockSpec((_BLK, 1), lambda i: (i, 0)),
        out_shape=jax.ShapeDtypeStruct((_N, 1), jnp.float32),
    )(h, parts, Wh, Wm, bn, Wo, bo)


def _sc_body(mall_hbm, src_hbm, dst_hbm, zeros_hbm, out_hbm,
             src_v, dst_v, rows_a, rows_b, acc, sem_a, sem_b, sem_z):
    c = lax.axis_index("c")
    s = lax.axis_index("s")
    pltpu.async_copy(zeros_hbm.at[pl.ds(s * _RPT, _RPT)],
                     acc.at[pl.ds(s * _RPT, _RPT)], sem_z)

    for p in range(_K // _KH):
        pltpu.sync_copy(src_hbm.at[c, s, pl.ds(p * _KH, _KH)], src_v)
        pltpu.sync_copy(dst_hbm.at[c, s, pl.ds(p * _KH, _KH)], dst_v)
        pltpu.async_copy(mall_hbm.at[src_v.at[0]], rows_a, sem_a)
        pltpu.async_copy(mall_hbm.at[src_v.at[1]], rows_b, sem_b)
        if p == 0:
            pltpu.make_async_copy(zeros_hbm.at[pl.ds(s * _RPT, _RPT)],
                                  acc.at[pl.ds(s * _RPT, _RPT)],
                                  sem_z).wait()
            plsc.subcore_barrier()

        def body(gg, carry):
            g = gg * 2
            pltpu.make_async_copy(mall_hbm.at[src_v.at[g]], rows_a,
                                  sem_a).wait()
            pltpu.sync_copy(rows_a, acc.at[dst_v.at[g]], add=True)

            @pl.when(g + 2 < _KH)
            def _():
                pltpu.async_copy(mall_hbm.at[src_v.at[g + 2]], rows_a, sem_a)

            pltpu.make_async_copy(mall_hbm.at[src_v.at[g + 1]], rows_b,
                                  sem_b).wait()
            pltpu.sync_copy(rows_b, acc.at[dst_v.at[g + 1]], add=True)

            @pl.when(g + 3 < _KH)
            def _():
                pltpu.async_copy(mall_hbm.at[src_v.at[g + 3]], rows_b, sem_b)

            return carry

        lax.fori_loop(0, _KH // 2, body, 0)
    plsc.subcore_barrier()
    pltpu.sync_copy(acc.at[pl.ds(s * _RPT, _RPT)],
                    out_hbm.at[c, pl.ds(s * _RPT, _RPT)])


def _sc_segsum(m_all, src, dst, zeros):
    mesh = plsc.VectorSubcoreMesh(core_axis_name="c", subcore_axis_name="s")
    f = functools.partial(
        pl.kernel,
        out_type=jax.ShapeDtypeStruct((_NC, _NP, _D), jnp.float32),
        mesh=mesh,
        scratch_types=[
            pltpu.VMEM((_KH, _C), jnp.int32),
            pltpu.VMEM((_KH, _C), jnp.int32),
            pltpu.VMEM((_C, _D), jnp.float32),
            pltpu.VMEM((_C, _D), jnp.float32),
            pltpu.VMEM_SHARED((_NP, _D), jnp.float32),
            pltpu.SemaphoreType.DMA,
            pltpu.SemaphoreType.DMA,
            pltpu.SemaphoreType.DMA,
        ],
    )(_sc_body)
    return f(m_all, src, dst, zeros)


def kernel(x, edge_index, W_in, b_in, W_e1, b_e1, W_e2, b_e2, W_n, b_n,
           W_out, b_out):
    pad = _EP - _E
    pad_ids = lax.iota(jnp.int32, pad)
    src = jnp.concatenate(
        [edge_index[0], pad_ids % _N]).reshape(_NC, _NS, _K, _C)
    dst = jnp.concatenate(
        [edge_index[1], _N + pad_ids % (_NP - _N)]).reshape(
            _NC, _NS, _K, _C)
    zeros = jnp.zeros((_NP, _D), jnp.float32)
    b_in2 = b_in.reshape(1, _D)
    b_e12 = b_e1.reshape(1, _D)
    b_e22 = b_e2.reshape(1, _D)
    b_n2 = b_n.reshape(1, _D)
    b_out2 = b_out.reshape(1, 1)
    W_nh = W_n[:_D]
    W_nm = W_n[_D:]

    h, m_all = _in_edge(x, W_in, b_in2, W_e1, b_e12, W_e2, b_e22)
    parts = _sc_segsum(m_all, src, dst, zeros)
    h, m_all = _node_edge(h, parts, W_nh, W_nm, b_n2,
                          W_e1, b_e12, W_e2, b_e22)
    parts = _sc_segsum(m_all, src, dst, zeros)
    return _node_out(h, parts, W_nh, W_nm, b_n2, W_out, b_out2)
